# baseline (speedup 1.0000x reference)
import functools

import jax
import jax.numpy as jnp
from jax import lax
from jax.experimental import pallas as pl
from jax.experimental.pallas import tpu as pltpu
from jax.experimental.pallas import tpu_sc as plsc

NC = 2
NS = 16
NW = NC * NS
CH = 128
NPASS = 2


@functools.lru_cache(maxsize=None)
def _sc_gather(B, K, W, V2):
    b_per_w = B // NW
    n_chunks = (b_per_w * K) // CH
    tp_chunks = b_per_w // CH
    pass_chunks = n_chunks // NPASS
    acc_rows = 2 * b_per_w // NPASS

    mesh = plsc.VectorSubcoreMesh(core_axis_name="c", subcore_axis_name="s")
    NBUF = 2
    LAG = 1

    @functools.partial(
        pl.kernel,
        out_type=(jax.ShapeDtypeStruct((B, W), jnp.float32),
                  jax.ShapeDtypeStruct((B, W), jnp.float32),
                  jax.ShapeDtypeStruct((2 * B, W), jnp.float32)),
        mesh=mesh,
        scratch_types=[
            pltpu.VMEM((n_chunks, CH), jnp.int32),
            pltpu.VMEM((n_chunks, CH), jnp.int32),
            pltpu.VMEM_SHARED((NS * acc_rows, W), jnp.float32),
            [pltpu.VMEM((CH, W), jnp.float32) for _ in range(NBUF)],
            pltpu.VMEM((tp_chunks, CH), jnp.int32),
            pltpu.VMEM((tp_chunks, CH), jnp.int32),
            [pltpu.SemaphoreType.DMA for _ in range(NBUF)],
            [pltpu.SemaphoreType.DMA for _ in range(NBUF)],
        ],
    )
    def gather_kernel(tgt_hbm, ctx_hbm, tidx_hbm, pidx_hbm, nidx_hbm,
                      didx_hbm, zeros_hbm, t_out, p_out, nw_out,
                      nidx_v, didx_v, acc_v, bufs, tidx_v, pidx_v,
                      gsems, osems):
        sid = lax.axis_index("s")
        wid = sid * NC + lax.axis_index("c")
        base_b = wid * b_per_w
        base_sh = sid * acc_rows

        pltpu.sync_copy(tidx_hbm.at[pl.ds(wid * tp_chunks, tp_chunks)],
                        tidx_v)
        pltpu.sync_copy(pidx_hbm.at[pl.ds(wid * tp_chunks, tp_chunks)],
                        pidx_v)
        pltpu.sync_copy(nidx_hbm.at[pl.ds(wid * n_chunks, n_chunks)], nidx_v)
        pltpu.sync_copy(didx_hbm.at[pl.ds(wid * n_chunks, n_chunks)], didx_v)
        pltpu.sync_copy(zeros_hbm, acc_v.at[pl.ds(base_sh, acc_rows)])

        work = [("t", c, False) for c in range(tp_chunks)]
        work += [("p", c, False) for c in range(tp_chunks)]
        for ps in range(NPASS):
            for i in range(pass_chunks):
                c = ps * pass_chunks + i
                work.append(("n", c, i == pass_chunks - 1))
        n_items = len(work)

        def issue_gather(kind, c, b):
            if kind == "t":
                return pltpu.async_copy(
                    tgt_hbm.at[tidx_v.at[c]], bufs[b], gsems[b])
            if kind == "p":
                return pltpu.async_copy(
                    ctx_hbm.at[pidx_v.at[c]], bufs[b], gsems[b])
            return pltpu.async_copy(
                ctx_hbm.at[nidx_v.at[c]], bufs[b], gsems[b])

        def issue_out(kind, c, b):
            if kind == "t":
                return pltpu.async_copy(
                    bufs[b], t_out.at[pl.ds(base_b + c * CH, CH)], osems[b])
            if kind == "p":
                return pltpu.async_copy(
                    bufs[b], p_out.at[pl.ds(base_b + c * CH, CH)], osems[b])
            return pltpu.async_copy(
                bufs[b], acc_v.at[didx_v.at[c]], osems[b], add=True)

        gdescs = [None] * NBUF
        odescs = [None] * NBUF
        issued = []
        for step in range(n_items + LAG):
            if step < n_items:
                b = step % NBUF
                if step >= NBUF and odescs[b] is not None:
                    odescs[b].wait()
                    odescs[b] = None
                kind, c, _ = work[step]
                gdescs[b] = issue_gather(kind, c, b)
            d = step - LAG
            if 0 <= d < n_items:
                b = d % NBUF
                gdescs[b].wait()
                kind, c, flush = work[d]
                odescs[b] = issue_out(kind, c, b)
                if flush:
                    for bb in range(NBUF):
                        if odescs[bb] is not None:
                            odescs[bb].wait()
                            odescs[bb] = None
                    ps = (c + 1) // pass_chunks - 1
                    pltpu.sync_copy(
                        acc_v.at[pl.ds(base_sh, acc_rows)],
                        nw_out.at[pl.ds(2 * base_b + ps * acc_rows,
                                        acc_rows)])
                    if ps + 1 < NPASS:
                        pltpu.sync_copy(
                            zeros_hbm, acc_v.at[pl.ds(base_sh, acc_rows)])
        for bb in range(NBUF):
            if odescs[bb] is not None:
                odescs[bb].wait()

    return gather_kernel


def _fold_body(lo_ref, hi_ref, o_ref):
    o_ref[:, :lo_ref.shape[1]] = lo_ref[...]
    o_ref[:, lo_ref.shape[1]:] = hi_ref[...]


@functools.lru_cache(maxsize=None)
def _fold(V, D):
    G = 125
    rows = V // 2 // G
    return pl.pallas_call(
        _fold_body,
        grid=(G,),
        in_specs=[
            pl.BlockSpec((rows, D), lambda i: (i, 0)),
            pl.BlockSpec((rows, D), lambda i: (i + G, 0)),
        ],
        out_specs=pl.BlockSpec((rows, 2 * D), lambda i: (i, 0)),
        out_shape=jax.ShapeDtypeStruct((V // 2, 2 * D), jnp.float32),
    )


def _loss_body(t2_ref, p2_ref, nw_ref, tpar_ref, ppar_ref, o_ref, B):
    _, W = t2_ref.shape
    D = W // 2
    t2 = t2_ref[...]
    p2 = p2_ref[...]
    tpar = tpar_ref[...]
    ppar = ppar_ref[...]
    t = t2[:, :D] + tpar * (t2[:, D:] - t2[:, :D])
    p = p2[:, :D] + ppar * (p2[:, D:] - p2[:, :D])
    nw = nw_ref[...]
    nsum = nw[:, :D] + nw[:, W + D:]

    pos = jnp.sum(t * p, axis=1)
    neg = jnp.sum(t * nsum, axis=1)

    def log_sigmoid(x):
        return jnp.minimum(x, 0.0) - jnp.log1p(jnp.exp(-jnp.abs(x)))

    part = -jnp.sum(log_sigmoid(pos) + log_sigmoid(-neg)) / B

    @pl.when(pl.program_id(0) == 0)
    def _():
        o_ref[0, 0] = 0.0

    o_ref[0, 0] += part


def kernel(target_embeddings, context_embeddings, target_block,
           positive_context_block, negative_context_blocks):
    V, D = target_embeddings.shape
    B = target_block.shape[0]
    K = negative_context_blocks.shape[1]
    b_per_w = B // NW
    W = 2 * D

    H = V // 2
    fold = _fold(V, D)
    ctx2 = fold(context_embeddings, context_embeddings)
    tgt2 = fold(target_embeddings, target_embeddings)

    tb = target_block.astype(jnp.int32)
    pb = positive_context_block.astype(jnp.int32)
    nb = negative_context_blocks.astype(jnp.int32).reshape(-1)

    tidx = (tb % H).reshape(-1, CH)
    pidx = (pb % H).reshape(-1, CH)
    nidx = (nb % H).reshape(-1, CH)

    g = jnp.arange(B * K, dtype=jnp.int32)
    rows_per_pass = b_per_w // NPASS
    didx = (((g // (b_per_w * K)) // NC) * (2 * rows_per_pass)
            + 2 * ((g // K) % rows_per_pass)
            + (nb // H)).reshape(-1, CH)

    zeros = jnp.zeros((2 * b_per_w // NPASS, W), jnp.float32)

    t2, p2, nw = _sc_gather(B, K, W, V // 2)(
        tgt2, ctx2, tidx, pidx, nidx, didx, zeros)

    tpar = (tb // H).astype(jnp.float32).reshape(B, 1)
    ppar = (pb // H).astype(jnp.float32).reshape(B, 1)
    nw2 = nw.reshape(B, 2 * W)

    G = 8
    blk = B // G
    loss = pl.pallas_call(
        functools.partial(_loss_body, B=B),
        grid=(G,),
        in_specs=[
            pl.BlockSpec((blk, W), lambda i: (i, 0)),
            pl.BlockSpec((blk, W), lambda i: (i, 0)),
            pl.BlockSpec((blk, 2 * W), lambda i: (i, 0)),
            pl.BlockSpec((blk, 1), lambda i: (i, 0)),
            pl.BlockSpec((blk, 1), lambda i: (i, 0)),
        ],
        out_shape=jax.ShapeDtypeStruct((1, 1), jnp.float32),
        out_specs=pl.BlockSpec((1, 1), lambda i: (0, 0),
                               memory_space=pltpu.SMEM),
    )(t2, p2, nw2, tpar, ppar)
    return loss[0, 0]


# speedup vs baseline: 1.2348x; 1.2348x over previous
import functools

import jax
import jax.numpy as jnp
from jax import lax
from jax.experimental import pallas as pl
from jax.experimental.pallas import tpu as pltpu
from jax.experimental.pallas import tpu_sc as plsc

NC = 2
NS = 16
NW = NC * NS
CH = 128
NPASS = 2


@functools.lru_cache(maxsize=None)
def _sc_gather(B, K, W, V2):
    b_per_w = B // NW
    n_chunks = (b_per_w * K) // CH
    tp_chunks = b_per_w // CH
    pass_chunks = n_chunks // NPASS
    acc_rows = 2 * b_per_w // NPASS

    mesh = plsc.VectorSubcoreMesh(core_axis_name="c", subcore_axis_name="s")
    NBUF = 2
    LAG = 1

    @functools.partial(
        pl.kernel,
        out_type=(jax.ShapeDtypeStruct((B, W), jnp.float32),
                  jax.ShapeDtypeStruct((B, W), jnp.float32),
                  jax.ShapeDtypeStruct((2 * B, W), jnp.float32)),
        mesh=mesh,
        scratch_types=[
            pltpu.VMEM((n_chunks, CH), jnp.int32),
            pltpu.VMEM((n_chunks, CH), jnp.int32),
            pltpu.VMEM_SHARED((NS * acc_rows, W), jnp.float32),
            [pltpu.VMEM((CH, W), jnp.float32) for _ in range(NBUF)],
            pltpu.VMEM((tp_chunks, CH), jnp.int32),
            pltpu.VMEM((tp_chunks, CH), jnp.int32),
            [pltpu.SemaphoreType.DMA for _ in range(NBUF)],
            [pltpu.SemaphoreType.DMA for _ in range(NBUF)],
        ],
    )
    def gather_kernel(tgt_hbm, ctx_hbm, tidx_hbm, pidx_hbm, nidx_hbm,
                      didx_hbm, zeros_hbm, t_out, p_out, nw_out,
                      nidx_v, didx_v, acc_v, bufs, tidx_v, pidx_v,
                      gsems, osems):
        sid = lax.axis_index("s")
        wid = sid * NC + lax.axis_index("c")
        base_b = wid * b_per_w
        base_sh = sid * acc_rows

        pltpu.sync_copy(tidx_hbm.at[pl.ds(wid * tp_chunks, tp_chunks)],
                        tidx_v)
        pltpu.sync_copy(pidx_hbm.at[pl.ds(wid * tp_chunks, tp_chunks)],
                        pidx_v)
        pltpu.sync_copy(nidx_hbm.at[pl.ds(wid * n_chunks, n_chunks)], nidx_v)
        pltpu.sync_copy(didx_hbm.at[pl.ds(wid * n_chunks, n_chunks)], didx_v)
        pltpu.sync_copy(zeros_hbm, acc_v.at[pl.ds(base_sh, acc_rows)])

        work = [("t", c, False) for c in range(tp_chunks)]
        work += [("p", c, False) for c in range(tp_chunks)]
        for ps in range(NPASS):
            for i in range(pass_chunks):
                c = ps * pass_chunks + i
                work.append(("n", c, i == pass_chunks - 1))
        n_items = len(work)

        def issue_gather(kind, c, b):
            if kind == "t":
                return pltpu.async_copy(
                    tgt_hbm.at[tidx_v.at[c]], bufs[b], gsems[b])
            if kind == "p":
                return pltpu.async_copy(
                    ctx_hbm.at[pidx_v.at[c]], bufs[b], gsems[b])
            return pltpu.async_copy(
                ctx_hbm.at[nidx_v.at[c]], bufs[b], gsems[b])

        def issue_out(kind, c, b):
            if kind == "t":
                return pltpu.async_copy(
                    bufs[b], t_out.at[pl.ds(base_b + c * CH, CH)], osems[b])
            if kind == "p":
                return pltpu.async_copy(
                    bufs[b], p_out.at[pl.ds(base_b + c * CH, CH)], osems[b])
            return pltpu.async_copy(
                bufs[b], acc_v.at[didx_v.at[c]], osems[b], add=True)

        gdescs = [None] * NBUF
        odescs = [None] * NBUF
        issued = []
        for step in range(n_items + LAG):
            if step < n_items:
                b = step % NBUF
                if step >= NBUF and odescs[b] is not None:
                    odescs[b].wait()
                    odescs[b] = None
                kind, c, _ = work[step]
                gdescs[b] = issue_gather(kind, c, b)
            d = step - LAG
            if 0 <= d < n_items:
                b = d % NBUF
                gdescs[b].wait()
                kind, c, flush = work[d]
                odescs[b] = issue_out(kind, c, b)
                if flush:
                    for bb in range(NBUF):
                        if odescs[bb] is not None:
                            odescs[bb].wait()
                            odescs[bb] = None
                    ps = (c + 1) // pass_chunks - 1
                    pltpu.sync_copy(
                        acc_v.at[pl.ds(base_sh, acc_rows)],
                        nw_out.at[pl.ds(2 * base_b + ps * acc_rows,
                                        acc_rows)])
                    if ps + 1 < NPASS:
                        pltpu.sync_copy(
                            zeros_hbm, acc_v.at[pl.ds(base_sh, acc_rows)])
        for bb in range(NBUF):
            if odescs[bb] is not None:
                odescs[bb].wait()

    return gather_kernel


def _fold_body(lo_ref, hi_ref, o_ref):
    o_ref[:, :lo_ref.shape[1]] = lo_ref[...]
    o_ref[:, lo_ref.shape[1]:] = hi_ref[...]


def _fold3_body(lo_ref, hi_ref, o_ref):
    o_ref[:, :lo_ref.shape[2]] = lo_ref[:, 0, :]
    o_ref[:, lo_ref.shape[2]:] = hi_ref[:, 0, :]


@functools.lru_cache(maxsize=None)
def _fold3(V, D):
    G = 125
    rows = V // 2 // G
    return pl.pallas_call(
        _fold3_body,
        grid=(G,),
        in_specs=[
            pl.BlockSpec((rows, 1, D), lambda i: (i, 0, 0)),
            pl.BlockSpec((rows, 1, D), lambda i: (i + G, 0, 0)),
        ],
        out_specs=pl.BlockSpec((rows, 2 * D), lambda i: (i, 0)),
        out_shape=jax.ShapeDtypeStruct((V // 2, 2 * D), jnp.float32),
    )


@functools.lru_cache(maxsize=None)
def _fold(V, D):
    G = 125
    rows = V // 2 // G
    return pl.pallas_call(
        _fold_body,
        grid=(G,),
        in_specs=[
            pl.BlockSpec((rows, D), lambda i: (i, 0)),
            pl.BlockSpec((rows, D), lambda i: (i + G, 0)),
        ],
        out_specs=pl.BlockSpec((rows, 2 * D), lambda i: (i, 0)),
        out_shape=jax.ShapeDtypeStruct((V // 2, 2 * D), jnp.float32),
    )


def _loss_body(t2_ref, p2_ref, nw_ref, tpar_ref, ppar_ref, o_ref, B):
    _, W = t2_ref.shape
    D = W // 2
    t2 = t2_ref[...]
    p2 = p2_ref[...]
    tpar = tpar_ref[...]
    ppar = ppar_ref[...]
    t = t2[:, :D] + tpar * (t2[:, D:] - t2[:, :D])
    p = p2[:, :D] + ppar * (p2[:, D:] - p2[:, :D])
    nw = nw_ref[...]
    nsum = nw[:, :D] + nw[:, W + D:]

    pos = jnp.sum(t * p, axis=1)
    neg = jnp.sum(t * nsum, axis=1)

    def log_sigmoid(x):
        return jnp.minimum(x, 0.0) - jnp.log1p(jnp.exp(-jnp.abs(x)))

    part = -jnp.sum(log_sigmoid(pos) + log_sigmoid(-neg)) / B

    @pl.when(pl.program_id(0) == 0)
    def _():
        o_ref[0, 0] = 0.0

    o_ref[0, 0] += part


def kernel(target_embeddings, context_embeddings, target_block,
           positive_context_block, negative_context_blocks):
    V, D = target_embeddings.shape
    B = target_block.shape[0]
    K = negative_context_blocks.shape[1]
    b_per_w = B // NW
    W = 2 * D

    H = V // 2
    fold = _fold3(V, D)
    ctx3 = context_embeddings.reshape(V, 1, D)
    tgt3 = target_embeddings.reshape(V, 1, D)
    ctx2 = fold(ctx3, ctx3)
    tgt2 = fold(tgt3, tgt3)

    tb = target_block.astype(jnp.int32)
    pb = positive_context_block.astype(jnp.int32)
    nb = negative_context_blocks.astype(jnp.int32).reshape(-1)

    tidx = (tb % H).reshape(-1, CH)
    pidx = (pb % H).reshape(-1, CH)
    nidx = (nb % H).reshape(-1, CH)

    g = jnp.arange(B * K, dtype=jnp.int32)
    rows_per_pass = b_per_w // NPASS
    didx = (((g // (b_per_w * K)) // NC) * (2 * rows_per_pass)
            + 2 * ((g // K) % rows_per_pass)
            + (nb // H)).reshape(-1, CH)

    zeros = jnp.zeros((2 * b_per_w // NPASS, W), jnp.float32)

    t2, p2, nw = _sc_gather(B, K, W, V // 2)(
        tgt2, ctx2, tidx, pidx, nidx, didx, zeros)

    tpar = (tb // H).astype(jnp.float32).reshape(B, 1)
    ppar = (pb // H).astype(jnp.float32).reshape(B, 1)
    nw2 = nw.reshape(B, 2 * W)

    G = 8
    blk = B // G
    loss = pl.pallas_call(
        functools.partial(_loss_body, B=B),
        grid=(G,),
        in_specs=[
            pl.BlockSpec((blk, W), lambda i: (i, 0)),
            pl.BlockSpec((blk, W), lambda i: (i, 0)),
            pl.BlockSpec((blk, 2 * W), lambda i: (i, 0)),
            pl.BlockSpec((blk, 1), lambda i: (i, 0)),
            pl.BlockSpec((blk, 1), lambda i: (i, 0)),
        ],
        out_shape=jax.ShapeDtypeStruct((1, 1), jnp.float32),
        out_specs=pl.BlockSpec((1, 1), lambda i: (0, 0),
                               memory_space=pltpu.SMEM),
    )(t2, p2, nw2, tpar, ppar)
    return loss[0, 0]


# speedup vs baseline: 1.3053x; 1.0571x over previous
import functools

import jax
import jax.numpy as jnp
from jax import lax
from jax.experimental import pallas as pl
from jax.experimental.pallas import tpu as pltpu
from jax.experimental.pallas import tpu_sc as plsc

NC = 2
NS = 16
NW = NC * NS
CH = 128
NPASS = 2


@functools.lru_cache(maxsize=None)
def _sc_gather(B, K, W, V2):
    b_per_w = B // NW
    n_chunks = (b_per_w * K) // CH
    tp_chunks = b_per_w // CH
    pass_chunks = n_chunks // NPASS
    acc_rows = 2 * b_per_w // NPASS

    mesh = plsc.VectorSubcoreMesh(core_axis_name="c", subcore_axis_name="s")
    NBUF = 2
    LAG = 1

    @functools.partial(
        pl.kernel,
        out_type=(jax.ShapeDtypeStruct((B, W), jnp.float32),
                  jax.ShapeDtypeStruct((2 * B, W), jnp.float32)),
        mesh=mesh,
        scratch_types=[
            pltpu.VMEM((n_chunks, CH), jnp.int32),
            pltpu.VMEM((n_chunks, CH), jnp.int32),
            pltpu.VMEM_SHARED((NS * acc_rows, W), jnp.float32),
            [pltpu.VMEM((CH, W), jnp.float32) for _ in range(NBUF)],
            pltpu.VMEM((tp_chunks, CH), jnp.int32),
            [pltpu.SemaphoreType.DMA for _ in range(NBUF)],
            [pltpu.SemaphoreType.DMA for _ in range(NBUF)],
        ],
    )
    def gather_kernel(ctx_hbm, pidx_hbm, nidx_hbm,
                      didx_hbm, zeros_hbm, p_out, nw_out,
                      nidx_v, didx_v, acc_v, bufs, pidx_v,
                      gsems, osems):
        sid = lax.axis_index("s")
        wid = sid * NC + lax.axis_index("c")
        base_b = wid * b_per_w
        base_sh = sid * acc_rows

        pltpu.sync_copy(pidx_hbm.at[pl.ds(wid * tp_chunks, tp_chunks)],
                        pidx_v)
        pltpu.sync_copy(nidx_hbm.at[pl.ds(wid * n_chunks, n_chunks)], nidx_v)
        pltpu.sync_copy(didx_hbm.at[pl.ds(wid * n_chunks, n_chunks)], didx_v)
        pltpu.sync_copy(zeros_hbm, acc_v.at[pl.ds(base_sh, acc_rows)])

        work = [("p", c, False) for c in range(tp_chunks)]
        for ps in range(NPASS):
            for i in range(pass_chunks):
                c = ps * pass_chunks + i
                work.append(("n", c, i == pass_chunks - 1))
        n_items = len(work)

        def issue_gather(kind, c, b):
            if kind == "p":
                return pltpu.async_copy(
                    ctx_hbm.at[pidx_v.at[c]], bufs[b], gsems[b])
            return pltpu.async_copy(
                ctx_hbm.at[nidx_v.at[c]], bufs[b], gsems[b])

        def issue_out(kind, c, b):
            if kind == "p":
                return pltpu.async_copy(
                    bufs[b], p_out.at[pl.ds(base_b + c * CH, CH)], osems[b])
            return pltpu.async_copy(
                bufs[b], acc_v.at[didx_v.at[c]], osems[b], add=True)

        gdescs = [None] * NBUF
        odescs = [None] * NBUF
        issued = []
        for step in range(n_items + LAG):
            if step < n_items:
                b = step % NBUF
                if step >= NBUF and odescs[b] is not None:
                    odescs[b].wait()
                    odescs[b] = None
                kind, c, _ = work[step]
                gdescs[b] = issue_gather(kind, c, b)
            d = step - LAG
            if 0 <= d < n_items:
                b = d % NBUF
                gdescs[b].wait()
                kind, c, flush = work[d]
                odescs[b] = issue_out(kind, c, b)
                if flush:
                    for bb in range(NBUF):
                        if odescs[bb] is not None:
                            odescs[bb].wait()
                            odescs[bb] = None
                    ps = (c + 1) // pass_chunks - 1
                    pltpu.sync_copy(
                        acc_v.at[pl.ds(base_sh, acc_rows)],
                        nw_out.at[pl.ds(2 * base_b + ps * acc_rows,
                                        acc_rows)])
                    if ps + 1 < NPASS:
                        pltpu.sync_copy(
                            zeros_hbm, acc_v.at[pl.ds(base_sh, acc_rows)])
        for bb in range(NBUF):
            if odescs[bb] is not None:
                odescs[bb].wait()

    return gather_kernel


@functools.lru_cache(maxsize=None)
def _sc_gather_t(B, W, V2):
    b_per_w = B // NW
    tp_chunks = b_per_w // CH
    mesh = plsc.VectorSubcoreMesh(core_axis_name="c", subcore_axis_name="s")

    @functools.partial(
        pl.kernel,
        out_type=jax.ShapeDtypeStruct((B, W), jnp.float32),
        mesh=mesh,
        scratch_types=[
            pltpu.VMEM((tp_chunks, CH), jnp.int32),
            [pltpu.VMEM((CH, W), jnp.float32) for _ in range(2)],
            [pltpu.SemaphoreType.DMA for _ in range(2)],
            [pltpu.SemaphoreType.DMA for _ in range(2)],
        ],
    )
    def t_kernel(tgt_hbm, tidx_hbm, t_out, tidx_v, bufs, gsems, osems):
        wid = lax.axis_index("s") * NC + lax.axis_index("c")
        base_b = wid * b_per_w
        pltpu.sync_copy(tidx_hbm.at[pl.ds(wid * tp_chunks, tp_chunks)],
                        tidx_v)
        gdescs = [None, None]
        odescs = [None, None]
        for c in range(tp_chunks + 1):
            if c < tp_chunks:
                b = c % 2
                if odescs[b] is not None:
                    odescs[b].wait()
                gdescs[b] = pltpu.async_copy(
                    tgt_hbm.at[tidx_v.at[c]], bufs[b], gsems[b])
            d = c - 1
            if d >= 0:
                b = d % 2
                gdescs[b].wait()
                odescs[b] = pltpu.async_copy(
                    bufs[b], t_out.at[pl.ds(base_b + d * CH, CH)], osems[b])
        for b in range(2):
            if odescs[b] is not None:
                odescs[b].wait()

    return t_kernel


def _fold_body(lo_ref, hi_ref, o_ref):
    o_ref[:, :lo_ref.shape[1]] = lo_ref[...]
    o_ref[:, lo_ref.shape[1]:] = hi_ref[...]


def _fold3_body(lo_ref, hi_ref, o_ref):
    o_ref[:, :lo_ref.shape[2]] = lo_ref[:, 0, :]
    o_ref[:, lo_ref.shape[2]:] = hi_ref[:, 0, :]


@functools.lru_cache(maxsize=None)
def _fold3(V, D):
    G = 125
    rows = V // 2 // G
    return pl.pallas_call(
        _fold3_body,
        grid=(G,),
        in_specs=[
            pl.BlockSpec((rows, 1, D), lambda i: (i, 0, 0)),
            pl.BlockSpec((rows, 1, D), lambda i: (i + G, 0, 0)),
        ],
        out_specs=pl.BlockSpec((rows, 2 * D), lambda i: (i, 0)),
        out_shape=jax.ShapeDtypeStruct((V // 2, 2 * D), jnp.float32),
    )


@functools.lru_cache(maxsize=None)
def _fold(V, D):
    G = 125
    rows = V // 2 // G
    return pl.pallas_call(
        _fold_body,
        grid=(G,),
        in_specs=[
            pl.BlockSpec((rows, D), lambda i: (i, 0)),
            pl.BlockSpec((rows, D), lambda i: (i + G, 0)),
        ],
        out_specs=pl.BlockSpec((rows, 2 * D), lambda i: (i, 0)),
        out_shape=jax.ShapeDtypeStruct((V // 2, 2 * D), jnp.float32),
    )


def _loss_body(t2_ref, p2_ref, nw_ref, tpar_ref, ppar_ref, o_ref, B):
    _, W = t2_ref.shape
    D = W // 2
    t2 = t2_ref[...]
    p2 = p2_ref[...]
    tpar = tpar_ref[...]
    ppar = ppar_ref[...]
    t = t2[:, :D] + tpar * (t2[:, D:] - t2[:, :D])
    p = p2[:, :D] + ppar * (p2[:, D:] - p2[:, :D])
    nw = nw_ref[...]
    nsum = nw[:, :D] + nw[:, W + D:]

    pos = jnp.sum(t * p, axis=1)
    neg = jnp.sum(t * nsum, axis=1)

    def log_sigmoid(x):
        return jnp.minimum(x, 0.0) - jnp.log1p(jnp.exp(-jnp.abs(x)))

    part = -jnp.sum(log_sigmoid(pos) + log_sigmoid(-neg)) / B

    @pl.when(pl.program_id(0) == 0)
    def _():
        o_ref[0, 0] = 0.0

    o_ref[0, 0] += part


def kernel(target_embeddings, context_embeddings, target_block,
           positive_context_block, negative_context_blocks):
    V, D = target_embeddings.shape
    B = target_block.shape[0]
    K = negative_context_blocks.shape[1]
    b_per_w = B // NW
    W = 2 * D

    H = V // 2
    fold = _fold3(V, D)
    ctx3 = context_embeddings.reshape(V, 1, D)
    tgt3 = target_embeddings.reshape(V, 1, D)
    ctx2 = fold(ctx3, ctx3)
    tgt2 = fold(tgt3, tgt3)

    tb = target_block.astype(jnp.int32)
    pb = positive_context_block.astype(jnp.int32)
    nb = negative_context_blocks.astype(jnp.int32).reshape(-1)

    tidx = (tb % H).reshape(-1, CH)
    pidx = (pb % H).reshape(-1, CH)
    nidx = (nb % H).reshape(-1, CH)

    g = jnp.arange(B * K, dtype=jnp.int32)
    rows_per_pass = b_per_w // NPASS
    didx = (((g // (b_per_w * K)) // NC) * (2 * rows_per_pass)
            + 2 * ((g // K) % rows_per_pass)
            + (nb // H)).reshape(-1, CH)

    zeros = jnp.zeros((2 * b_per_w // NPASS, W), jnp.float32)

    p2, nw = _sc_gather(B, K, W, V // 2)(
        ctx2, pidx, nidx, didx, zeros)
    t2 = _sc_gather_t(B, W, V // 2)(tgt2, tidx)

    tpar = (tb // H).astype(jnp.float32).reshape(B, 1)
    ppar = (pb // H).astype(jnp.float32).reshape(B, 1)
    nw2 = nw.reshape(B, 2 * W)

    G = 8
    blk = B // G
    loss = pl.pallas_call(
        functools.partial(_loss_body, B=B),
        grid=(G,),
        in_specs=[
            pl.BlockSpec((blk, W), lambda i: (i, 0)),
            pl.BlockSpec((blk, W), lambda i: (i, 0)),
            pl.BlockSpec((blk, 2 * W), lambda i: (i, 0)),
            pl.BlockSpec((blk, 1), lambda i: (i, 0)),
            pl.BlockSpec((blk, 1), lambda i: (i, 0)),
        ],
        out_shape=jax.ShapeDtypeStruct((1, 1), jnp.float32),
        out_specs=pl.BlockSpec((1, 1), lambda i: (0, 0),
                               memory_space=pltpu.SMEM),
    )(t2, p2, nw2, tpar, ppar)
    return loss[0, 0]


# speedup vs baseline: 1.3635x; 1.0446x over previous
import functools

import jax
import jax.numpy as jnp
from jax import lax
from jax.experimental import pallas as pl
from jax.experimental.pallas import tpu as pltpu
from jax.experimental.pallas import tpu_sc as plsc

NC = 2
NS = 16
NW = NC * NS
CH = 128
NPASS = 2


@functools.lru_cache(maxsize=None)
def _sc_gather(B, K, W, V2):
    b_per_w = B // NW
    n_chunks = (b_per_w * K) // CH
    tp_chunks = b_per_w // CH
    pass_chunks = n_chunks // NPASS
    acc_rows = 2 * b_per_w // NPASS

    mesh = plsc.VectorSubcoreMesh(core_axis_name="c", subcore_axis_name="s")
    NBUF = 2
    LAG = 1

    @functools.partial(
        pl.kernel,
        out_type=(jax.ShapeDtypeStruct((B, W), jnp.float32),
                  jax.ShapeDtypeStruct((2 * B, W), jnp.float32)),
        mesh=mesh,
        scratch_types=[
            pltpu.VMEM((n_chunks, CH), jnp.int32),
            pltpu.VMEM((n_chunks, CH), jnp.int32),
            pltpu.VMEM_SHARED((NS * acc_rows, W), jnp.float32),
            [pltpu.VMEM((CH, W), jnp.float32) for _ in range(NBUF)],
            pltpu.VMEM((tp_chunks, CH), jnp.int32),
            [pltpu.SemaphoreType.DMA for _ in range(NBUF)],
            [pltpu.SemaphoreType.DMA for _ in range(NBUF)],
        ],
    )
    def gather_kernel(ctx_hbm, pidx_hbm, nidx_hbm,
                      didx_hbm, zeros_hbm, p_out, nw_out,
                      nidx_v, didx_v, acc_v, bufs, pidx_v,
                      gsems, osems):
        sid = lax.axis_index("s")
        wid = sid * NC + lax.axis_index("c")
        base_b = wid * b_per_w
        base_sh = sid * acc_rows

        pltpu.sync_copy(pidx_hbm.at[pl.ds(wid * tp_chunks, tp_chunks)],
                        pidx_v)
        pltpu.sync_copy(nidx_hbm.at[pl.ds(wid * n_chunks, n_chunks)], nidx_v)
        pltpu.sync_copy(didx_hbm.at[pl.ds(wid * n_chunks, n_chunks)], didx_v)
        pltpu.sync_copy(zeros_hbm, acc_v.at[pl.ds(base_sh, acc_rows)])

        work = [("p", c, False) for c in range(tp_chunks)]
        for ps in range(NPASS):
            for i in range(pass_chunks):
                c = ps * pass_chunks + i
                work.append(("n", c, i == pass_chunks - 1))
        n_items = len(work)

        def issue_gather(kind, c, b):
            if kind == "p":
                return pltpu.async_copy(
                    ctx_hbm.at[pidx_v.at[c]], bufs[b], gsems[b])
            return pltpu.async_copy(
                ctx_hbm.at[nidx_v.at[c]], bufs[b], gsems[b])

        def issue_out(kind, c, b):
            if kind == "p":
                return pltpu.async_copy(
                    bufs[b], p_out.at[pl.ds(base_b + c * CH, CH)], osems[b])
            return pltpu.async_copy(
                bufs[b], acc_v.at[didx_v.at[c]], osems[b], add=True)

        gdescs = [None] * NBUF
        odescs = [None] * NBUF
        for step in range(n_items + LAG):
            if step < n_items:
                b = step % NBUF
                if step >= NBUF and odescs[b] is not None:
                    odescs[b].wait()
                    odescs[b] = None
                kind, c, _ = work[step]
                gdescs[b] = issue_gather(kind, c, b)
            d = step - LAG
            if 0 <= d < n_items:
                b = d % NBUF
                gdescs[b].wait()
                kind, c, flush = work[d]
                odescs[b] = issue_out(kind, c, b)
                if flush:
                    for bb in range(NBUF):
                        if odescs[bb] is not None:
                            odescs[bb].wait()
                            odescs[bb] = None
                    ps = (c + 1) // pass_chunks - 1
                    pltpu.sync_copy(
                        acc_v.at[pl.ds(base_sh, acc_rows)],
                        nw_out.at[pl.ds(2 * base_b + ps * acc_rows,
                                        acc_rows)])
                    if ps + 1 < NPASS:
                        pltpu.sync_copy(
                            zeros_hbm, acc_v.at[pl.ds(base_sh, acc_rows)])
        for bb in range(NBUF):
            if odescs[bb] is not None:
                odescs[bb].wait()

    return gather_kernel


@functools.lru_cache(maxsize=None)
def _sc_gather_t(B, W, V2):
    b_per_w = B // NW
    tp_chunks = b_per_w // CH
    mesh = plsc.VectorSubcoreMesh(core_axis_name="c", subcore_axis_name="s")

    @functools.partial(
        pl.kernel,
        out_type=jax.ShapeDtypeStruct((B, W), jnp.float32),
        mesh=mesh,
        scratch_types=[
            pltpu.VMEM((tp_chunks, CH), jnp.int32),
            [pltpu.VMEM((CH, W), jnp.float32) for _ in range(2)],
            [pltpu.SemaphoreType.DMA for _ in range(2)],
            [pltpu.SemaphoreType.DMA for _ in range(2)],
        ],
    )
    def t_kernel(tgt_hbm, tidx_hbm, t_out, tidx_v, bufs, gsems, osems):
        wid = lax.axis_index("s") * NC + lax.axis_index("c")
        base_b = wid * b_per_w
        pltpu.sync_copy(tidx_hbm.at[pl.ds(wid * tp_chunks, tp_chunks)],
                        tidx_v)
        gdescs = [None, None]
        odescs = [None, None]
        for c in range(tp_chunks + 1):
            if c < tp_chunks:
                b = c % 2
                if odescs[b] is not None:
                    odescs[b].wait()
                gdescs[b] = pltpu.async_copy(
                    tgt_hbm.at[tidx_v.at[c]], bufs[b], gsems[b])
            d = c - 1
            if d >= 0:
                b = d % 2
                gdescs[b].wait()
                odescs[b] = pltpu.async_copy(
                    bufs[b], t_out.at[pl.ds(base_b + d * CH, CH)], osems[b])
        for b in range(2):
            if odescs[b] is not None:
                odescs[b].wait()

    return t_kernel


def _fold3_body(lo_ref, hi_ref, o_ref):
    o_ref[:, :lo_ref.shape[2]] = lo_ref[:, 0, :]
    o_ref[:, lo_ref.shape[2]:] = hi_ref[:, 0, :]


@functools.lru_cache(maxsize=None)
def _fold3(V, D):
    G = 50
    rows = V // 2 // G
    return pl.pallas_call(
        _fold3_body,
        grid=(G,),
        in_specs=[
            pl.BlockSpec((rows, 1, D), lambda i: (i, 0, 0)),
            pl.BlockSpec((rows, 1, D), lambda i: (i + G, 0, 0)),
        ],
        out_specs=pl.BlockSpec((rows, 2 * D), lambda i: (i, 0)),
        out_shape=jax.ShapeDtypeStruct((V // 2, 2 * D), jnp.float32),
    )


def _loss_body(t2_ref, p2_ref, nw_ref, tpar_ref, ppar_ref, o_ref, B):
    _, W = t2_ref.shape
    D = W // 2
    t2 = t2_ref[...]
    p2 = p2_ref[...]
    tpar = tpar_ref[...]
    ppar = ppar_ref[...]
    t = t2[:, :D] + tpar * (t2[:, D:] - t2[:, :D])
    p = p2[:, :D] + ppar * (p2[:, D:] - p2[:, :D])
    nw = nw_ref[...]
    nsum = nw[:, :D] + nw[:, W + D:]

    pos = jnp.sum(t * p, axis=1)
    neg = jnp.sum(t * nsum, axis=1)

    def log_sigmoid(x):
        return jnp.minimum(x, 0.0) - jnp.log1p(jnp.exp(-jnp.abs(x)))

    part = -jnp.sum(log_sigmoid(pos) + log_sigmoid(-neg)) / B

    @pl.when(pl.program_id(0) == 0)
    def _():
        o_ref[0, 0] = 0.0

    o_ref[0, 0] += part


def kernel(target_embeddings, context_embeddings, target_block,
           positive_context_block, negative_context_blocks):
    V, D = target_embeddings.shape
    B = target_block.shape[0]
    K = negative_context_blocks.shape[1]
    b_per_w = B // NW
    W = 2 * D

    H = V // 2
    fold = _fold3(V, D)
    ctx3 = context_embeddings.reshape(V, 1, D)
    tgt3 = target_embeddings.reshape(V, 1, D)
    ctx2 = fold(ctx3, ctx3)
    tgt2 = fold(tgt3, tgt3)

    tb = target_block.astype(jnp.int32)
    pb = positive_context_block.astype(jnp.int32)
    nb = negative_context_blocks.astype(jnp.int32).reshape(-1)

    tidx = (tb % H).reshape(-1, CH)
    pidx = (pb % H).reshape(-1, CH)
    nidx = (nb % H).reshape(-1, CH)

    g = jnp.arange(B * K, dtype=jnp.int32)
    rows_per_pass = b_per_w // NPASS
    didx = (((g // (b_per_w * K)) // NC) * (2 * rows_per_pass)
            + 2 * ((g // K) % rows_per_pass)
            + (nb // H)).reshape(-1, CH)

    zeros = jnp.zeros((2 * b_per_w // NPASS, W), jnp.float32)

    p2, nw = _sc_gather(B, K, W, V // 2)(
        ctx2, pidx, nidx, didx, zeros)
    t2 = _sc_gather_t(B, W, V // 2)(tgt2, tidx)

    tpar = (tb // H).astype(jnp.float32).reshape(B, 1)
    ppar = (pb // H).astype(jnp.float32).reshape(B, 1)
    nw2 = nw.reshape(B, 2 * W)

    G = 8
    blk = B // G
    loss = pl.pallas_call(
        functools.partial(_loss_body, B=B),
        grid=(G,),
        in_specs=[
            pl.BlockSpec((blk, W), lambda i: (i, 0)),
            pl.BlockSpec((blk, W), lambda i: (i, 0)),
            pl.BlockSpec((blk, 2 * W), lambda i: (i, 0)),
            pl.BlockSpec((blk, 1), lambda i: (i, 0)),
            pl.BlockSpec((blk, 1), lambda i: (i, 0)),
        ],
        out_shape=jax.ShapeDtypeStruct((1, 1), jnp.float32),
        out_specs=pl.BlockSpec((1, 1), lambda i: (0, 0),
                               memory_space=pltpu.SMEM),
    )(t2, p2, nw2, tpar, ppar)
    return loss[0, 0]


# speedup vs baseline: 1.3748x; 1.0083x over previous
import functools

import jax
import jax.numpy as jnp
from jax import lax
from jax.experimental import pallas as pl
from jax.experimental.pallas import tpu as pltpu
from jax.experimental.pallas import tpu_sc as plsc

NC = 2
NS = 16
NW = NC * NS
CH = 128
NPASS = 2


@functools.lru_cache(maxsize=None)
def _sc_gather(B, K, W, V2):
    b_per_w = B // NW
    n_chunks = (b_per_w * K) // CH
    tp_chunks = b_per_w // CH
    pass_chunks = n_chunks // NPASS
    acc_rows = 2 * b_per_w // NPASS

    mesh = plsc.VectorSubcoreMesh(core_axis_name="c", subcore_axis_name="s")
    NBUF = 2
    LAG = 1

    @functools.partial(
        pl.kernel,
        out_type=(jax.ShapeDtypeStruct((B, W), jnp.float32),
                  jax.ShapeDtypeStruct((2 * B, W), jnp.float32)),
        mesh=mesh,
        scratch_types=[
            pltpu.VMEM((n_chunks, CH), jnp.int32),
            pltpu.VMEM((n_chunks, CH), jnp.int32),
            pltpu.VMEM_SHARED((NS * acc_rows, W), jnp.float32),
            [pltpu.VMEM((CH, W), jnp.float32) for _ in range(NBUF)],
            pltpu.VMEM((tp_chunks, CH), jnp.int32),
            [pltpu.SemaphoreType.DMA for _ in range(NBUF)],
            [pltpu.SemaphoreType.DMA for _ in range(NBUF)],
        ],
    )
    def gather_kernel(ctx_hbm, pidx_hbm, nidx_hbm,
                      didx_hbm, zeros_hbm, p_out, nw_out,
                      nidx_v, didx_v, acc_v, bufs, pidx_v,
                      gsems, osems):
        sid = lax.axis_index("s")
        wid = sid * NC + lax.axis_index("c")
        base_b = wid * b_per_w
        base_sh = sid * acc_rows

        pltpu.sync_copy(pidx_hbm.at[pl.ds(wid * tp_chunks, tp_chunks)],
                        pidx_v)
        pltpu.sync_copy(nidx_hbm.at[pl.ds(wid * n_chunks, n_chunks)], nidx_v)
        pltpu.sync_copy(didx_hbm.at[pl.ds(wid * n_chunks, n_chunks)], didx_v)
        pltpu.sync_copy(zeros_hbm, acc_v.at[pl.ds(base_sh, acc_rows)])

        work = [("p", c, False) for c in range(tp_chunks)]
        for ps in range(NPASS):
            for i in range(pass_chunks):
                c = ps * pass_chunks + i
                work.append(("n", c, i == pass_chunks - 1))
        n_items = len(work)

        def issue_gather(kind, c, b):
            if kind == "p":
                return pltpu.async_copy(
                    ctx_hbm.at[pidx_v.at[c]], bufs[b], gsems[b])
            return pltpu.async_copy(
                ctx_hbm.at[nidx_v.at[c]], bufs[b], gsems[b])

        def issue_out(kind, c, b):
            if kind == "p":
                return pltpu.async_copy(
                    bufs[b], p_out.at[pl.ds(base_b + c * CH, CH)], osems[b])
            return pltpu.async_copy(
                bufs[b], acc_v.at[didx_v.at[c]], osems[b], add=True)

        gdescs = [None] * NBUF
        odescs = [None] * NBUF
        for step in range(n_items + LAG):
            if step < n_items:
                b = step % NBUF
                if step >= NBUF and odescs[b] is not None:
                    odescs[b].wait()
                    odescs[b] = None
                kind, c, _ = work[step]
                gdescs[b] = issue_gather(kind, c, b)
            d = step - LAG
            if 0 <= d < n_items:
                b = d % NBUF
                gdescs[b].wait()
                kind, c, flush = work[d]
                odescs[b] = issue_out(kind, c, b)
                if flush:
                    for bb in range(NBUF):
                        if odescs[bb] is not None:
                            odescs[bb].wait()
                            odescs[bb] = None
                    ps = (c + 1) // pass_chunks - 1
                    pltpu.sync_copy(
                        acc_v.at[pl.ds(base_sh, acc_rows)],
                        nw_out.at[pl.ds(2 * base_b + ps * acc_rows,
                                        acc_rows)])
                    if ps + 1 < NPASS:
                        pltpu.sync_copy(
                            zeros_hbm, acc_v.at[pl.ds(base_sh, acc_rows)])
        for bb in range(NBUF):
            if odescs[bb] is not None:
                odescs[bb].wait()

    return gather_kernel


@functools.lru_cache(maxsize=None)
def _sc_gather_t(B, W, V2):
    b_per_w = B // NW
    tp_chunks = b_per_w // CH
    mesh = plsc.VectorSubcoreMesh(core_axis_name="c", subcore_axis_name="s")

    @functools.partial(
        pl.kernel,
        out_type=jax.ShapeDtypeStruct((B, W), jnp.float32),
        mesh=mesh,
        scratch_types=[
            pltpu.VMEM((tp_chunks, CH), jnp.int32),
            [pltpu.VMEM((CH, W), jnp.float32) for _ in range(2)],
            [pltpu.SemaphoreType.DMA for _ in range(2)],
            [pltpu.SemaphoreType.DMA for _ in range(2)],
        ],
    )
    def t_kernel(tgt_hbm, tidx_hbm, t_out, tidx_v, bufs, gsems, osems):
        wid = lax.axis_index("s") * NC + lax.axis_index("c")
        base_b = wid * b_per_w
        pltpu.sync_copy(tidx_hbm.at[pl.ds(wid * tp_chunks, tp_chunks)],
                        tidx_v)
        gdescs = [None, None]
        odescs = [None, None]
        for c in range(tp_chunks + 1):
            if c < tp_chunks:
                b = c % 2
                if odescs[b] is not None:
                    odescs[b].wait()
                gdescs[b] = pltpu.async_copy(
                    tgt_hbm.at[tidx_v.at[c]], bufs[b], gsems[b])
            d = c - 1
            if d >= 0:
                b = d % 2
                gdescs[b].wait()
                odescs[b] = pltpu.async_copy(
                    bufs[b], t_out.at[pl.ds(base_b + d * CH, CH)], osems[b])
        for b in range(2):
            if odescs[b] is not None:
                odescs[b].wait()

    return t_kernel


def _fold3_body(lo_ref, hi_ref, o_ref):
    o_ref[:, :lo_ref.shape[2]] = lo_ref[:, 0, :]
    o_ref[:, lo_ref.shape[2]:] = hi_ref[:, 0, :]


@functools.lru_cache(maxsize=None)
def _fold3(V, D):
    G = 25
    rows = V // 2 // G
    return pl.pallas_call(
        _fold3_body,
        grid=(G,),
        in_specs=[
            pl.BlockSpec((rows, 1, D), lambda i: (i, 0, 0)),
            pl.BlockSpec((rows, 1, D), lambda i: (i + G, 0, 0)),
        ],
        out_specs=pl.BlockSpec((rows, 2 * D), lambda i: (i, 0)),
        out_shape=jax.ShapeDtypeStruct((V // 2, 2 * D), jnp.float32),
    )


def _loss_body(t2_ref, p2_ref, nw_ref, tpar_ref, ppar_ref, o_ref, B):
    _, W = t2_ref.shape
    D = W // 2
    t2 = t2_ref[...]
    p2 = p2_ref[...]
    tpar = tpar_ref[...]
    ppar = ppar_ref[...]
    t = t2[:, :D] + tpar * (t2[:, D:] - t2[:, :D])
    p = p2[:, :D] + ppar * (p2[:, D:] - p2[:, :D])
    nw = nw_ref[...]
    nsum = nw[:, :D] + nw[:, W + D:]

    pos = jnp.sum(t * p, axis=1)
    neg = jnp.sum(t * nsum, axis=1)

    def log_sigmoid(x):
        return jnp.minimum(x, 0.0) - jnp.log1p(jnp.exp(-jnp.abs(x)))

    part = -jnp.sum(log_sigmoid(pos) + log_sigmoid(-neg)) / B

    @pl.when(pl.program_id(0) == 0)
    def _():
        o_ref[0, 0] = 0.0

    o_ref[0, 0] += part


def kernel(target_embeddings, context_embeddings, target_block,
           positive_context_block, negative_context_blocks):
    V, D = target_embeddings.shape
    B = target_block.shape[0]
    K = negative_context_blocks.shape[1]
    b_per_w = B // NW
    W = 2 * D

    H = V // 2
    fold = _fold3(V, D)
    ctx3 = context_embeddings.reshape(V, 1, D)
    tgt3 = target_embeddings.reshape(V, 1, D)
    ctx2 = fold(ctx3, ctx3)
    tgt2 = fold(tgt3, tgt3)

    tb = target_block.astype(jnp.int32)
    pb = positive_context_block.astype(jnp.int32)
    nb = negative_context_blocks.astype(jnp.int32).reshape(-1)

    tidx = (tb % H).reshape(-1, CH)
    pidx = (pb % H).reshape(-1, CH)
    nidx = (nb % H).reshape(-1, CH)

    g = jnp.arange(B * K, dtype=jnp.int32)
    rows_per_pass = b_per_w // NPASS
    didx = (((g // (b_per_w * K)) // NC) * (2 * rows_per_pass)
            + 2 * ((g // K) % rows_per_pass)
            + (nb // H)).reshape(-1, CH)

    zeros = jnp.zeros((2 * b_per_w // NPASS, W), jnp.float32)

    p2, nw = _sc_gather(B, K, W, V // 2)(
        ctx2, pidx, nidx, didx, zeros)
    t2 = _sc_gather_t(B, W, V // 2)(tgt2, tidx)

    tpar = (tb // H).astype(jnp.float32).reshape(B, 1)
    ppar = (pb // H).astype(jnp.float32).reshape(B, 1)
    nw2 = nw.reshape(B, 2 * W)

    G = 8
    blk = B // G
    loss = pl.pallas_call(
        functools.partial(_loss_body, B=B),
        grid=(G,),
        in_specs=[
            pl.BlockSpec((blk, W), lambda i: (i, 0)),
            pl.BlockSpec((blk, W), lambda i: (i, 0)),
            pl.BlockSpec((blk, 2 * W), lambda i: (i, 0)),
            pl.BlockSpec((blk, 1), lambda i: (i, 0)),
            pl.BlockSpec((blk, 1), lambda i: (i, 0)),
        ],
        out_shape=jax.ShapeDtypeStruct((1, 1), jnp.float32),
        out_specs=pl.BlockSpec((1, 1), lambda i: (0, 0),
                               memory_space=pltpu.SMEM),
    )(t2, p2, nw2, tpar, ppar)
    return loss[0, 0]
